# baseline (speedup 1.0000x reference)
import functools

import jax
import jax.numpy as jnp
from jax.experimental import pallas as pl


def _tc_body(x_ref, g_ref, nz_ref, o_ref):
    s = jnp.sqrt(
        jax.lax.dot(x_ref[...], g_ref[...], preferred_element_type=jnp.float32)
    )
    o_ref[...] = jnp.exp(nz_ref[...] * s)


@functools.partial(jax.jit, static_argnames=("interpret",))
def kernel(diffs, center_idx, zetas, interpret=False):
    B, E, C, F = diffs.shape
    S = center_idx.shape[0]
    n_rows = B * E
    x2 = diffs.reshape(n_rows // 2, 2 * C * F)

    pos = center_idx * F + (F - 1)
    g64 = (pos[None, :] == jnp.arange(C * F, dtype=jnp.int32)[:, None]).astype(
        jnp.float32
    )
    zero = jnp.zeros_like(g64)
    g128 = jnp.block([[g64, zero], [zero, g64]])
    nz128 = jnp.concatenate([-zetas, -zetas]).reshape(1, 2 * S)

    BLK = 1024
    grid = (x2.shape[0] // BLK,)
    out2 = pl.pallas_call(
        _tc_body,
        grid=grid,
        in_specs=[
            pl.BlockSpec((BLK, 2 * C * F), lambda i: (i, 0)),
            pl.BlockSpec((2 * C * F, 2 * S), lambda i: (0, 0)),
            pl.BlockSpec((1, 2 * S), lambda i: (0, 0)),
        ],
        out_specs=pl.BlockSpec((BLK, 2 * S), lambda i: (i, 0)),
        out_shape=jax.ShapeDtypeStruct((x2.shape[0], 2 * S), jnp.float32),
        interpret=interpret,
    )(x2, g128, nz128)
    return out2.reshape(B, E, S)


# speedup vs baseline: 16.9163x; 16.9163x over previous
import functools

import jax
import jax.numpy as jnp
from jax.experimental import pallas as pl


def _tc_body(x_ref, g_ref, nz_ref, o_ref):
    blk, e, cf = x_ref.shape
    xb = x_ref[...].reshape(blk * e, cf)
    s = jnp.sqrt(jax.lax.dot(xb, g_ref[...], preferred_element_type=jnp.float32))
    o_ref[...] = jnp.exp(nz_ref[...] * s).reshape(blk, e, g_ref.shape[1])


@functools.partial(jax.jit, static_argnames=("interpret",))
def kernel(diffs, center_idx, zetas, interpret=False):
    B, E, C, F = diffs.shape
    S = center_idx.shape[0]
    x3 = diffs.reshape(B, E, C * F)

    pos = center_idx * F + (F - 1)
    g64 = (pos[None, :] == jnp.arange(C * F, dtype=jnp.int32)[:, None]).astype(
        jnp.float32
    )
    nz = (-zetas).reshape(1, S)

    BLK = min(256, B)
    grid = (B // BLK,)
    out = pl.pallas_call(
        _tc_body,
        grid=grid,
        in_specs=[
            pl.BlockSpec((BLK, E, C * F), lambda i: (i, 0, 0)),
            pl.BlockSpec((C * F, S), lambda i: (0, 0)),
            pl.BlockSpec((1, S), lambda i: (0, 0)),
        ],
        out_specs=pl.BlockSpec((BLK, E, S), lambda i: (i, 0, 0)),
        out_shape=jax.ShapeDtypeStruct((B, E, S), jnp.float32),
        interpret=interpret,
    )(x3, g64, nz)
    return out


# speedup vs baseline: 99.6198x; 5.8890x over previous
import functools

import jax
import jax.numpy as jnp
from jax.experimental import pallas as pl


def _tc_body(x_ref, g_ref, nz_ref, o_ref):
    E, C, F, L = x_ref.shape
    S = o_ref.shape[1]
    d3 = x_ref[:, :, F - 1, :]
    nzc = nz_ref[:, 0:1]
    for e in range(E):
        r = jax.lax.dot(
            g_ref[...], d3[e], preferred_element_type=jnp.float32
        )
        o_ref[e] = jnp.exp(nzc * jnp.sqrt(r))


@functools.partial(jax.jit, static_argnames=("interpret",))
def kernel(diffs, center_idx, zetas, interpret=False):
    B, E, C, F = diffs.shape
    S = center_idx.shape[0]
    x_t = jnp.transpose(diffs, (1, 2, 3, 0))

    g = (center_idx[:, None] == jnp.arange(C, dtype=jnp.int32)[None, :]).astype(
        jnp.float32
    )
    nz = jnp.broadcast_to((-zetas)[:, None], (S, 128))

    BL = min(256, B)
    grid = (B // BL,)
    out_t = pl.pallas_call(
        _tc_body,
        grid=grid,
        in_specs=[
            pl.BlockSpec((E, C, F, BL), lambda i: (0, 0, 0, i)),
            pl.BlockSpec((S, C), lambda i: (0, 0)),
            pl.BlockSpec((S, 128), lambda i: (0, 0)),
        ],
        out_specs=pl.BlockSpec((E, S, BL), lambda i: (0, 0, i)),
        out_shape=jax.ShapeDtypeStruct((E, S, B), jnp.float32),
        interpret=interpret,
    )(x_t, g, nz)
    return jnp.transpose(out_t, (2, 0, 1))
